# baseline (speedup 1.0000x reference)
import functools

import jax
import jax.numpy as jnp
from jax import lax
from jax.experimental import pallas as pl
from jax.experimental.pallas import tpu as pltpu
from jax.experimental.pallas import tpu_sc as plsc

_NC = 2
_NS = 16
_LANES = 16
_NW = _NC * _NS
_NBUF = 4


@functools.lru_cache(maxsize=None)
def _build(N, D, W, chunk):
    rows_per_w = N // _NW
    n_chunks = rows_per_w // chunk
    vecs_per_row = D // _LANES
    assert n_chunks % _NBUF == 0 and n_chunks >= 2 * _NBUF

    mesh = plsc.VectorSubcoreMesh(core_axis_name="c", subcore_axis_name="s")

    buf_types = []
    for _ in range(_NBUF):
        buf_types.append(pltpu.VMEM((chunk, D), jnp.float32))
    for _ in range(_NBUF):
        buf_types.append(pltpu.VMEM((chunk, D), jnp.float32))
    sem_types = [pltpu.SemaphoreType.DMA] * (3 * _NBUF)

    @functools.partial(
        pl.kernel,
        out_type=jax.ShapeDtypeStruct((N, D), jnp.float32),
        mesh=mesh,
        scratch_types=[
            pltpu.VMEM((rows_per_w,), jnp.int32),
            pltpu.VMEM((rows_per_w,), jnp.int32),
            pltpu.VMEM((rows_per_w,), jnp.int32),
        ] + buf_types + sem_types,
    )
    def pe_add(x_hbm, i0_hbm, i1_hbm, table_hbm, out_hbm,
               i0_v, i1_v, fl_v, *bufs_and_sems):
        xb = bufs_and_sems[:_NBUF]
        rb = bufs_and_sems[_NBUF:2 * _NBUF]
        sx = bufs_and_sems[2 * _NBUF:3 * _NBUF]
        sg = bufs_and_sems[3 * _NBUF:4 * _NBUF]
        so = bufs_and_sems[4 * _NBUF:5 * _NBUF]

        wid = lax.axis_index("s") * _NC + lax.axis_index("c")
        base = wid * rows_per_w

        pltpu.sync_copy(i0_hbm.at[pl.ds(base, rows_per_w)], i0_v)
        pltpu.sync_copy(i1_hbm.at[pl.ds(base, rows_per_w)], i1_v)

        def flat_body(j, carry):
            s = pl.ds(j * _LANES, _LANES)
            fl_v[s] = i0_v[s] * W + i1_v[s]
            return carry

        lax.fori_loop(0, rows_per_w // _LANES, flat_body, 0)

        def issue(c, b):
            off = base + c * chunk
            pltpu.async_copy(x_hbm.at[pl.ds(off, chunk)], xb[b], sx[b])
            pltpu.async_copy(
                table_hbm.at[fl_v.at[pl.ds(c * chunk, chunk)]], rb[b], sg[b])

        def wait_in(c, b):
            off = base + c * chunk
            pltpu.make_async_copy(
                x_hbm.at[pl.ds(off, chunk)], xb[b], sx[b]).wait()
            pltpu.make_async_copy(
                table_hbm.at[fl_v.at[pl.ds(c * chunk, chunk)]],
                rb[b], sg[b]).wait()

        def drain_out(c, b):
            off = base + c * chunk
            pltpu.make_async_copy(
                xb[b], out_hbm.at[pl.ds(off, chunk)], so[b]).wait()

        for b in range(_NBUF - 1):
            issue(b, b)

        def step(i, carry):
            for b in range(_NBUF):
                c = _NBUF * i + b
                nb = (b + _NBUF - 1) % _NBUF

                @pl.when(c + _NBUF - 1 < n_chunks)
                def _():
                    @pl.when(c >= 1)
                    def _():
                        drain_out(c - 1, nb)
                    issue(c + _NBUF - 1, nb)

                wait_in(c, b)

                def add_body(r, carry2):
                    for k in range(vecs_per_row):
                        s = pl.ds(k * _LANES, _LANES)
                        plsc.addupdate(xb[b].at[r, s], rb[b][r, s])
                    return carry2

                lax.fori_loop(0, chunk, add_body, 0)

                off = base + c * chunk
                pltpu.async_copy(xb[b], out_hbm.at[pl.ds(off, chunk)], so[b])
            return carry

        lax.fori_loop(0, n_chunks // _NBUF, step, 0)
        for k in range(_NBUF):
            c = n_chunks - _NBUF + k
            drain_out(c, c % _NBUF)

    return pe_add


@functools.lru_cache(maxsize=None)
def _build_tc(N, D, W, V, n_sc, tb):
    n_tc = N - n_sc
    g0 = n_sc // tb
    assert n_sc % tb == 0 and n_tc % tb == 0

    def body(i0_ref, i1_ref, x_ref, tab_ref, o_ref):
        idx = i0_ref[0, 0, :] * W + i1_ref[0, 0, :]
        oh = (idx[:, None] ==
              lax.broadcasted_iota(jnp.int32, (tb, V), 1))
        pos = jax.lax.dot(oh.astype(jnp.bfloat16), tab_ref[...],
                          preferred_element_type=jnp.float32)
        o_ref[...] = x_ref[...] + pos

    return pl.pallas_call(
        body,
        grid=(n_tc // tb,),
        in_specs=[
            pl.BlockSpec((1, 1, tb), lambda g: (g0 + g, 0, 0)),
            pl.BlockSpec((1, 1, tb), lambda g: (g0 + g, 0, 0)),
            pl.BlockSpec((tb, D), lambda g: (g0 + g, 0)),
            pl.BlockSpec((V, D), lambda g: (0, 0)),
        ],
        out_specs=pl.BlockSpec((tb, D), lambda g: (g, 0)),
        out_shape=jax.ShapeDtypeStruct((n_tc, D), jnp.float32),
    )


@jax.jit
def kernel(x, patch_indices, positional_encoding):
    batch, num_patches, d = x.shape
    H, W, _ = positional_encoding.shape
    N = batch * num_patches
    V = H * W
    n_sc = 18432
    tb = 512

    xf = x.reshape(N, d)
    table = positional_encoding.reshape(V, d)
    i0 = patch_indices[:, :, 0].astype(jnp.int32).reshape(N)
    i1 = patch_indices[:, :, 1].astype(jnp.int32).reshape(N)
    i0b = i0.reshape(N // tb, 1, tb)
    i1b = i1.reshape(N // tb, 1, tb)

    out_sc = _build(n_sc, d, W, 16)(xf, i0, i1, table)
    out_tc = _build_tc(N, d, W, V, n_sc, tb)(
        i0b, i1b, xf, table.astype(jnp.bfloat16))
    out = jnp.concatenate([out_sc, out_tc], axis=0)
    return out.reshape(batch, num_patches, d)


# speedup vs baseline: 1.3391x; 1.3391x over previous
import functools

import jax
import jax.numpy as jnp
from jax import lax
from jax.experimental import pallas as pl
from jax.experimental.pallas import tpu as pltpu
from jax.experimental.pallas import tpu_sc as plsc

_NC = 2
_NS = 16
_LANES = 16
_NW = _NC * _NS
_NBUF = 4


@functools.lru_cache(maxsize=None)
def _build(N, D, W, chunk):
    rows_per_w = N // _NW
    n_chunks = rows_per_w // chunk
    vecs_per_row = D // _LANES
    assert n_chunks % _NBUF == 0 and n_chunks >= 2 * _NBUF

    mesh = plsc.VectorSubcoreMesh(core_axis_name="c", subcore_axis_name="s")

    buf_types = []
    for _ in range(_NBUF):
        buf_types.append(pltpu.VMEM((chunk, D), jnp.float32))
    for _ in range(_NBUF):
        buf_types.append(pltpu.VMEM((chunk, D), jnp.float32))
    sem_types = [pltpu.SemaphoreType.DMA] * (3 * _NBUF)

    @functools.partial(
        pl.kernel,
        out_type=jax.ShapeDtypeStruct((N, D), jnp.float32),
        mesh=mesh,
        scratch_types=[
            pltpu.VMEM((rows_per_w,), jnp.int32),
            pltpu.VMEM((rows_per_w,), jnp.int32),
            pltpu.VMEM((rows_per_w,), jnp.int32),
        ] + buf_types + sem_types,
    )
    def pe_add(x_hbm, i0_hbm, i1_hbm, table_hbm, out_hbm,
               i0_v, i1_v, fl_v, *bufs_and_sems):
        xb = bufs_and_sems[:_NBUF]
        rb = bufs_and_sems[_NBUF:2 * _NBUF]
        sx = bufs_and_sems[2 * _NBUF:3 * _NBUF]
        sg = bufs_and_sems[3 * _NBUF:4 * _NBUF]
        so = bufs_and_sems[4 * _NBUF:5 * _NBUF]

        wid = lax.axis_index("s") * _NC + lax.axis_index("c")
        base = wid * rows_per_w

        for b in range(_NBUF - 1):
            pltpu.async_copy(
                x_hbm.at[pl.ds(base + b * chunk, chunk)], xb[b], sx[b])

        cp0 = pltpu.async_copy(
            i0_hbm.at[pl.ds(base, rows_per_w)], i0_v, so[0])
        cp1 = pltpu.async_copy(
            i1_hbm.at[pl.ds(base, rows_per_w)], i1_v, so[1])
        cp0.wait()
        cp1.wait()

        def flat_body(j, carry):
            s = pl.ds(j * _LANES, _LANES)
            fl_v[s] = i0_v[s] * W + i1_v[s]
            return carry

        lax.fori_loop(0, rows_per_w // _LANES, flat_body, 0)

        def issue(c, b):
            off = base + c * chunk
            pltpu.async_copy(x_hbm.at[pl.ds(off, chunk)], xb[b], sx[b])
            pltpu.async_copy(
                table_hbm.at[fl_v.at[pl.ds(c * chunk, chunk)]], rb[b], sg[b])

        def wait_in(c, b):
            off = base + c * chunk
            pltpu.make_async_copy(
                x_hbm.at[pl.ds(off, chunk)], xb[b], sx[b]).wait()
            pltpu.make_async_copy(
                table_hbm.at[fl_v.at[pl.ds(c * chunk, chunk)]],
                rb[b], sg[b]).wait()

        def drain_out(c, b):
            off = base + c * chunk
            pltpu.make_async_copy(
                xb[b], out_hbm.at[pl.ds(off, chunk)], so[b]).wait()

        for b in range(_NBUF - 1):
            pltpu.async_copy(
                table_hbm.at[fl_v.at[pl.ds(b * chunk, chunk)]], rb[b], sg[b])

        def step(i, carry):
            for b in range(_NBUF):
                c = _NBUF * i + b
                nb = (b + _NBUF - 1) % _NBUF

                @pl.when(c + _NBUF - 1 < n_chunks)
                def _():
                    @pl.when(c >= 1)
                    def _():
                        drain_out(c - 1, nb)
                    issue(c + _NBUF - 1, nb)

                wait_in(c, b)

                def add_body(r, carry2):
                    for k in range(vecs_per_row):
                        s = pl.ds(k * _LANES, _LANES)
                        plsc.addupdate(xb[b].at[r, s], rb[b][r, s])
                    return carry2

                lax.fori_loop(0, chunk, add_body, 0)

                off = base + c * chunk
                pltpu.async_copy(xb[b], out_hbm.at[pl.ds(off, chunk)], so[b])
            return carry

        lax.fori_loop(0, n_chunks // _NBUF, step, 0)
        for k in range(_NBUF):
            c = n_chunks - _NBUF + k
            drain_out(c, c % _NBUF)

    return pe_add


@jax.jit
def kernel(x, patch_indices, positional_encoding):
    batch, num_patches, d = x.shape
    H, W, _ = positional_encoding.shape
    N = batch * num_patches

    xf = x.reshape(N, d)
    table = positional_encoding.reshape(H * W, d)
    i0 = patch_indices[:, :, 0].astype(jnp.int32).reshape(N)
    i1 = patch_indices[:, :, 1].astype(jnp.int32).reshape(N)

    out = _build(N, d, W, 16)(xf, i0, i1, table)
    return out.reshape(batch, num_patches, d)


# speedup vs baseline: 1.3489x; 1.0073x over previous
import functools

import jax
import jax.numpy as jnp
from jax import lax
from jax.experimental import pallas as pl
from jax.experimental.pallas import tpu as pltpu
from jax.experimental.pallas import tpu_sc as plsc

_NC = 2
_NS = 16
_LANES = 16
_NW = _NC * _NS
_NBUF = 4


@functools.lru_cache(maxsize=None)
def _build(N, D, W, chunk):
    rows_per_w = N // _NW
    n_chunks = rows_per_w // chunk
    vecs_per_row = D // _LANES
    assert n_chunks % _NBUF == 0 and n_chunks >= 2 * _NBUF

    mesh = plsc.VectorSubcoreMesh(core_axis_name="c", subcore_axis_name="s")

    buf_types = []
    for _ in range(_NBUF):
        buf_types.append(pltpu.VMEM((chunk, D), jnp.float32))
    for _ in range(_NBUF):
        buf_types.append(pltpu.VMEM((chunk, D), jnp.float32))
    sem_types = [pltpu.SemaphoreType.DMA] * (3 * _NBUF)

    @functools.partial(
        pl.kernel,
        out_type=jax.ShapeDtypeStruct((N, D), jnp.float32),
        mesh=mesh,
        scratch_types=[
            pltpu.VMEM((rows_per_w,), jnp.int32),
            pltpu.VMEM((rows_per_w,), jnp.int32),
            pltpu.VMEM((rows_per_w,), jnp.int32),
        ] + buf_types + sem_types,
    )
    def pe_add(x_hbm, i0_hbm, i1_hbm, table_hbm, out_hbm,
               i0_v, i1_v, fl_v, *bufs_and_sems):
        xb = bufs_and_sems[:_NBUF]
        rb = bufs_and_sems[_NBUF:2 * _NBUF]
        sx = bufs_and_sems[2 * _NBUF:3 * _NBUF]
        sg = bufs_and_sems[3 * _NBUF:4 * _NBUF]
        so = bufs_and_sems[4 * _NBUF:5 * _NBUF]

        wid = lax.axis_index("s") * _NC + lax.axis_index("c")
        base = wid * rows_per_w

        pltpu.sync_copy(i0_hbm.at[pl.ds(base, rows_per_w)], i0_v)
        pltpu.sync_copy(i1_hbm.at[pl.ds(base, rows_per_w)], i1_v)

        def flat_body(j, carry):
            s = pl.ds(j * _LANES, _LANES)
            fl_v[s] = i0_v[s] * W + i1_v[s]
            return carry

        lax.fori_loop(0, rows_per_w // _LANES, flat_body, 0)

        def issue(c, b):
            off = base + c * chunk
            pltpu.async_copy(x_hbm.at[pl.ds(off, chunk)], xb[b], sx[b])
            pltpu.async_copy(
                table_hbm.at[fl_v.at[pl.ds(c * chunk, chunk)]], rb[b], sg[b])

        def wait_in(c, b):
            off = base + c * chunk
            pltpu.make_async_copy(
                x_hbm.at[pl.ds(off, chunk)], xb[b], sx[b]).wait()
            pltpu.make_async_copy(
                table_hbm.at[fl_v.at[pl.ds(c * chunk, chunk)]],
                rb[b], sg[b]).wait()

        def drain_out(c, b):
            off = base + c * chunk
            pltpu.make_async_copy(
                xb[b], out_hbm.at[pl.ds(off, chunk)], so[b]).wait()

        for b in range(_NBUF - 1):
            issue(b, b)

        def step(i, carry):
            for b in range(_NBUF):
                c = _NBUF * i + b
                nb = (b + _NBUF - 1) % _NBUF

                @pl.when(c + _NBUF - 1 < n_chunks)
                def _():
                    @pl.when(c >= 1)
                    def _():
                        drain_out(c - 1, nb)
                    issue(c + _NBUF - 1, nb)

                wait_in(c, b)

                def add_body(r, carry2):
                    for k in range(vecs_per_row):
                        s = pl.ds(k * _LANES, _LANES)
                        plsc.addupdate(xb[b].at[r, s], rb[b][r, s])
                    return carry2

                lax.fori_loop(0, chunk, add_body, 0)

                off = base + c * chunk
                pltpu.async_copy(xb[b], out_hbm.at[pl.ds(off, chunk)], so[b])
            return carry

        lax.fori_loop(0, n_chunks // _NBUF, step, 0)
        for k in range(_NBUF):
            c = n_chunks - _NBUF + k
            drain_out(c, c % _NBUF)

    return pe_add


@jax.jit
def kernel(x, patch_indices, positional_encoding):
    batch, num_patches, d = x.shape
    H, W, _ = positional_encoding.shape
    N = batch * num_patches

    xf = x.reshape(N, d)
    table = positional_encoding.reshape(H * W, d)
    i0 = patch_indices[:, :, 0].astype(jnp.int32).reshape(N)
    i1 = patch_indices[:, :, 1].astype(jnp.int32).reshape(N)

    out = _build(N, d, W, 16)(xf, i0, i1, table)
    return out.reshape(batch, num_patches, d)
